# Initial kernel scaffold; baseline (speedup 1.0000x reference)
import jax
import jax.numpy as jnp
from jax.experimental import pallas as pl


def kernel(x, W_in, b_in, W_out, b_out, We_0, be_0, Ws_0, bs_0, We_1, be_1, Ws_1, bs_1):
    raise NotImplementedError("write your pallas kernel here")


# speedup vs baseline: 5.3239x; 5.3239x over previous
import jax
import jax.numpy as jnp
from jax.experimental import pallas as pl
from jax.experimental.pallas import tpu as pltpu

C = 32
F = 256
H = 32
HH = H * H
P = C * C
CHUNK = 128
NCHUNK = HH // CHUNK


def _body(xT_ref, WinT_ref, bin_ref, WoutT_ref, bout_ref,
          We0c_0_ref, We1c_0_ref, bec_0_ref, WsT_0_ref, bsT_0_ref,
          We0c_1_ref, We1c_1_ref, bec_1_ref, WsT_1_ref, bsT_1_ref,
          yT_ref):
    xT = xT_ref[0]

    hT = jnp.maximum(
        jnp.dot(WinT_ref[...], xT, preferred_element_type=jnp.float32)
        + bin_ref[...], 0.0)

    iota_r = jax.lax.broadcasted_iota(jnp.int32, (C, P), 0)
    iota_p = jax.lax.broadcasted_iota(jnp.int32, (C, P), 1)
    R = (iota_p // C == iota_r).astype(jnp.float32)
    Rd = (iota_p % C == iota_r).astype(jnp.float32)
    iota_pp = jax.lax.broadcasted_iota(jnp.int32, (P, C), 0)
    iota_d = jax.lax.broadcasted_iota(jnp.int32, (P, C), 1)
    S = ((iota_pp % C == iota_d) &
         (iota_pp // C != iota_d)).astype(jnp.float32)

    hsrcT = jnp.dot(hT, R, preferred_element_type=jnp.float32)
    hdstT = jnp.dot(hT, Rd, preferred_element_type=jnp.float32)
    numf = jnp.sum(hsrcT * hdstT, axis=0, keepdims=True)
    nsrc = jnp.maximum(
        jnp.sqrt(jnp.sum(hsrcT * hsrcT, axis=0, keepdims=True)), 1e-8)
    ndst = jnp.maximum(
        jnp.sqrt(jnp.sum(hdstT * hdstT, axis=0, keepdims=True)), 1e-8)
    cosf = numf / (nsrc * ndst)
    dvec = hdstT - hsrcT
    distr = jnp.sqrt(jnp.sum(dvec * dvec, axis=0, keepdims=True))
    dmean = jnp.sum(distr) / float(C * (C - 1))
    distf = distr / (dmean + 1e-6)

    inv_deg = 1.0 / float(C - 1)

    for We0c, We1c, bec, WsT, bsT in (
            (We0c_0_ref, We1c_0_ref, bec_0_ref, WsT_0_ref, bsT_0_ref),
            (We0c_1_ref, We1c_1_ref, bec_1_ref, WsT_1_ref, bsT_1_ref)):
        hrepT = jnp.dot(hT, R, preferred_element_type=jnp.float32)
        msg_rows = []
        for c in range(NCHUNK):
            r0 = c * CHUNK
            pre = (We0c[r0:r0 + CHUNK] * cosf
                   + We1c[r0:r0 + CHUNK] * distf
                   + bec[r0:r0 + CHUNK])
            T = jnp.tanh(pre)
            for o in range(CHUNK // H):
                msg_rows.append(jnp.sum(T[o * H:(o + 1) * H] * hrepT,
                                        axis=0, keepdims=True))
        MSG = jnp.concatenate(msg_rows, axis=0)
        AGG = jnp.dot(MSG, S, preferred_element_type=jnp.float32)
        hT = jnp.maximum(
            AGG * inv_deg
            + jnp.dot(WsT[...], hT, preferred_element_type=jnp.float32)
            + bsT[...], 0.0)

    yT_ref[0] = xT + jnp.dot(WoutT_ref[...], hT,
                             preferred_element_type=jnp.float32) + bout_ref[...]


def kernel(x, W_in, b_in, W_out, b_out,
           We_0, be_0, Ws_0, bs_0, We_1, be_1, Ws_1, bs_1):
    B = x.shape[0]
    f32 = jnp.float32

    xT = x.transpose(0, 2, 1)
    WinT = W_in.T
    WoutT = W_out.T
    binT = b_in[:, None]
    boutT = b_out[:, None]

    def edge_cols(We, be):
        Wp = We.reshape(2, H, H).transpose(0, 2, 1).reshape(2, HH)
        bp = be.reshape(H, H).T.reshape(HH)
        return Wp[0][:, None], Wp[1][:, None], bp[:, None]

    We0c_0, We1c_0, bec_0 = edge_cols(We_0, be_0)
    We0c_1, We1c_1, bec_1 = edge_cols(We_1, be_1)
    WsT_0, bsT_0 = Ws_0.T, bs_0[:, None]
    WsT_1, bsT_1 = Ws_1.T, bs_1[:, None]

    full = lambda shape: pl.BlockSpec(shape, lambda b: (0,) * len(shape))
    grid_spec = pl.GridSpec(
        grid=(B,),
        in_specs=[
            pl.BlockSpec((1, F, C), lambda b: (b, 0, 0)),
            full((H, F)), full((H, 1)), full((F, H)), full((F, 1)),
            full((HH, 1)), full((HH, 1)), full((HH, 1)),
            full((H, H)), full((H, 1)),
            full((HH, 1)), full((HH, 1)), full((HH, 1)),
            full((H, H)), full((H, 1)),
        ],
        out_specs=pl.BlockSpec((1, F, C), lambda b: (b, 0, 0)),
    )
    yT = pl.pallas_call(
        _body,
        grid_spec=grid_spec,
        out_shape=jax.ShapeDtypeStruct((B, F, C), f32),
        compiler_params=pltpu.CompilerParams(
            dimension_semantics=("parallel",)),
    )(xT.astype(f32), WinT, binT, WoutT, boutT,
      We0c_0, We1c_0, bec_0, WsT_0, bsT_0,
      We0c_1, We1c_1, bec_1, WsT_1, bsT_1)
    return yT.transpose(0, 2, 1)
